# baseline (speedup 1.0000x reference)
import functools

import jax
import jax.numpy as jnp
from jax import lax
from jax.experimental import pallas as pl
from jax.experimental.pallas import tpu as pltpu
from jax.experimental.pallas import tpu_sc as plsc

BATCH = 16384
DIM = 32
NW = 32
B_PER_W = BATCH // NW
N_CHUNK = 4
CHUNK = B_PER_W // N_CHUNK
N_BLOCKS = B_PER_W // 16


def _body(xu_hbm, xv_hbm, u_hbm, v_hbm, out_hbm,
          idx_u, idx_v, rows_u, rows_v, out_v, sem):
  wid = lax.axis_index("s") * 2 + lax.axis_index("c")
  base = wid * B_PER_W

  pltpu.sync_copy(xu_hbm.at[pl.ds(wid * N_CHUNK, N_CHUNK)], idx_u)
  pltpu.sync_copy(xv_hbm.at[pl.ds(wid * N_CHUNK, N_CHUNK)], idx_v)

  copies = []
  for j in range(N_CHUNK):
    copies.append(pltpu.async_copy(
        u_hbm.at[idx_u.at[j]], rows_u.at[pl.ds(j * CHUNK, CHUNK)], sem))
    copies.append(pltpu.async_copy(
        v_hbm.at[idx_v.at[j]], rows_v.at[pl.ds(j * CHUNK, CHUNK)], sem))
  for c in copies:
    c.wait()

  iota = lax.iota(jnp.int32, 16)

  def block(b, _):
    rows = b * 16 + iota
    acc = jnp.zeros((16,), jnp.float32)
    for d in range(DIM):
      d_idx = jnp.full((16,), d, jnp.int32)
      ug = plsc.load_gather(rows_u, [rows, d_idx])
      vg = plsc.load_gather(rows_v, [rows, d_idx])
      acc = acc + ug * vg
    plsc.store_scatter(out_v, [rows], acc)
    return ()

  lax.fori_loop(0, N_BLOCKS, block, (), unroll=False)

  pltpu.sync_copy(out_v, out_hbm.at[pl.ds(base, B_PER_W)])


@functools.partial(
    pl.kernel,
    out_type=jax.ShapeDtypeStruct((BATCH,), jnp.float32),
    mesh=plsc.VectorSubcoreMesh(core_axis_name="c", subcore_axis_name="s"),
    compiler_params=pltpu.CompilerParams(
        needs_layout_passes=False, use_tc_tiling_on_sc=False),
    scratch_types=[
        pltpu.VMEM((N_CHUNK, CHUNK), jnp.int32),
        pltpu.VMEM((N_CHUNK, CHUNK), jnp.int32),
        pltpu.VMEM((B_PER_W, DIM), jnp.float32),
        pltpu.VMEM((B_PER_W, DIM), jnp.float32),
        pltpu.VMEM((B_PER_W,), jnp.float32),
        pltpu.SemaphoreType.DMA,
    ],
)
def _mf_sc(xu_hbm, xv_hbm, u_hbm, v_hbm, out_hbm,
           idx_u, idx_v, rows_u, rows_v, out_v, sem):
  _body(xu_hbm, xv_hbm, u_hbm, v_hbm, out_hbm,
        idx_u, idx_v, rows_u, rows_v, out_v, sem)


def kernel(x, U, V):
  xu = x[:, 0].reshape(BATCH // CHUNK, CHUNK)
  xv = x[:, 1].reshape(BATCH // CHUNK, CHUNK)
  return _mf_sc(xu, xv, U, V)
